# baseline (speedup 1.0000x reference)
import functools

import jax
import jax.numpy as jnp
from jax import lax
from jax.experimental import pallas as pl
from jax.experimental.pallas import tpu as pltpu
from jax.experimental.pallas import tpu_sc as plsc

E = 64
D = 768
F = 3072
N = 2048
K = 2
R = N * K

T = 256
NT = R // T
NI = NT + E
NF = 2
FC = F // NF

SC_CORES = 2
SC_SUBCORES = 16
SC_WORKERS = SC_CORES * SC_SUBCORES


def _gating_body(x_ref, gw_ref, e0_ref, e1_ref, w0_ref, w1_ref):
    x = x_ref[...]
    gw = gw_ref[...]
    logits = lax.dot_general(x, gw, (((1,), (1,)), ((), ())),
                             preferred_element_type=jnp.float32)
    col = lax.broadcasted_iota(jnp.int32, logits.shape, 1)
    m1 = jnp.max(logits, axis=1)
    a1 = jnp.min(jnp.where(logits == m1[:, None], col, E), axis=1)
    masked = jnp.where(col == a1[:, None], -1e30, logits)
    m2 = jnp.max(masked, axis=1)
    a2 = jnp.min(jnp.where(masked == m2[:, None], col, E), axis=1)
    e2 = jnp.exp(m2 - m1)
    denom = 1.0 + e2
    e0_ref[...] = a1[:, None]
    e1_ref[...] = a2[:, None]
    w0_ref[...] = (1.0 / denom)[:, None]
    w1_ref[...] = (e2 / denom)[:, None]


def _gating(x, gate_w, interpret=False):
    return pl.pallas_call(
        _gating_body,
        out_shape=(
            jax.ShapeDtypeStruct((N, 1), jnp.int32),
            jax.ShapeDtypeStruct((N, 1), jnp.int32),
            jax.ShapeDtypeStruct((N, 1), jnp.float32),
            jax.ShapeDtypeStruct((N, 1), jnp.float32),
        ),
        interpret=interpret,
    )(x, gate_w)


def _route_metadata(e0, e1, w0, w1):
    flat_e = jnp.concatenate([e0[:, 0], e1[:, 0]])
    flat_w = jnp.concatenate([w0[:, 0], w1[:, 0]])
    perm = jnp.argsort(flat_e, stable=True).astype(jnp.int32)
    tok = (perm % N).astype(jnp.int32)
    sorted_w = flat_w[perm][:, None]
    inv = jnp.zeros((R,), jnp.int32).at[perm].set(
        jnp.arange(R, dtype=jnp.int32))
    p0, p1 = inv[:N], inv[N:]

    counts = jnp.bincount(flat_e, length=E).astype(jnp.int32)
    ends = jnp.cumsum(counts).astype(jnp.int32)
    starts = ends - counts
    t0 = starts // T
    t1 = (ends + T - 1) // T
    nitems = jnp.where(counts > 0, t1 - t0, 0)
    item_cum = jnp.cumsum(nitems).astype(jnp.int32)
    total = item_cum[E - 1]

    i_idx = jnp.arange(NI, dtype=jnp.int32)
    e_i = jnp.searchsorted(item_cum, i_idx, side="right").astype(jnp.int32)
    valid = i_idx < total
    e_c = jnp.minimum(e_i, E - 1)
    prev = jnp.where(e_c > 0, item_cum[e_c - 1], 0)
    tile = (t0[e_c] + (i_idx - prev)).astype(jnp.int32)
    tile = jnp.where(valid, tile, NT - 1)
    grp = jnp.where(valid, e_c, flat_e[perm[R - 1]]).astype(jnp.int32)
    lo = jnp.clip(starts[e_c] - tile * T, 0, T)
    hi = jnp.clip(ends[e_c] - tile * T, 0, T)
    lo = jnp.where(valid, lo, 0).astype(jnp.int32)
    hi = jnp.where(valid, hi, 0).astype(jnp.int32)
    prev_tile = jnp.concatenate([jnp.full((1,), -1, jnp.int32), tile[:-1]])
    first = (valid & (tile != prev_tile)).astype(jnp.int32)
    return dict(tok=tok, sorted_w=sorted_w, p0=p0, p1=p1,
                grp=grp, tile=tile, lo=lo, hi=hi, first=first)


def _sc_dispatch(x, tok):
    bpw = R // SC_WORKERS
    mesh = plsc.VectorSubcoreMesh(core_axis_name="c", subcore_axis_name="s")

    @functools.partial(
        pl.kernel, mesh=mesh,
        out_type=jax.ShapeDtypeStruct((R, D), jnp.float32),
        scratch_types=[pltpu.VMEM((bpw,), jnp.int32),
                       pltpu.VMEM((bpw, D), jnp.float32),
                       pltpu.SemaphoreType.DMA],
    )
    def k(x_hbm, tok_hbm, out_hbm, idx_v, rows_v, sem):
        wid = lax.axis_index("s") * SC_CORES + lax.axis_index("c")
        base = wid * bpw
        pltpu.sync_copy(tok_hbm.at[pl.ds(base, bpw)], idx_v)
        pltpu.async_copy(x_hbm.at[idx_v], rows_v, sem).wait()
        pltpu.sync_copy(rows_v, out_hbm.at[pl.ds(base, bpw)])

    return k(x, tok)


def _gmm_body(grp_ref, tile_ref, lo_ref, hi_ref, first_ref,
              x_ref, w1_ref, w2_ref, sw_ref, out_ref):
    i = pl.program_id(0)
    j = pl.program_id(1)
    x = x_ref[...]
    h = lax.dot_general(x, w1_ref[0], (((1,), (1,)), ((), ())),
                        preferred_element_type=jnp.float32)
    h = 0.5 * h * (1.0 + lax.erf(h * 0.7071067811865476))
    z = lax.dot_general(h, w2_ref[0], (((1,), (1,)), ((), ())),
                        preferred_element_type=jnp.float32)
    z = z * sw_ref[...]
    rows = lax.broadcasted_iota(jnp.int32, (T, 1), 0)
    mask = (rows >= lo_ref[i]) & (rows < hi_ref[i])
    z = jnp.where(mask, z, 0.0)
    is_first = (first_ref[i] == 1) & (j == 0)

    @pl.when(is_first)
    def _():
        out_ref[...] = z

    @pl.when(jnp.logical_not(is_first))
    def _():
        out_ref[...] += z


def _gmm(sorted_x, W1, W2, sorted_w, grp, tile, lo, hi, first, interpret=False):
    grid_spec = pltpu.PrefetchScalarGridSpec(
        num_scalar_prefetch=5,
        grid=(NI, NF),
        in_specs=[
            pl.BlockSpec((T, D), lambda i, j, g, t, *_: (t[i], 0)),
            pl.BlockSpec((1, FC, D), lambda i, j, g, t, *_: (g[i], j, 0)),
            pl.BlockSpec((1, D, FC), lambda i, j, g, t, *_: (g[i], 0, j)),
            pl.BlockSpec((T, 1), lambda i, j, g, t, *_: (t[i], 0)),
        ],
        out_specs=pl.BlockSpec((T, D), lambda i, j, g, t, *_: (t[i], 0)),
    )
    return pl.pallas_call(
        _gmm_body,
        grid_spec=grid_spec,
        out_shape=jax.ShapeDtypeStruct((R, D), jnp.float32),
        compiler_params=pltpu.CompilerParams(
            dimension_semantics=("arbitrary", "arbitrary")),
        interpret=interpret,
    )(grp, tile, lo, hi, first, sorted_x, W1, W2, sorted_w)


def _sc_combine(z, p0, p1):
    bpw = N // SC_WORKERS
    mesh = plsc.VectorSubcoreMesh(core_axis_name="c", subcore_axis_name="s")

    @functools.partial(
        pl.kernel, mesh=mesh,
        out_type=jax.ShapeDtypeStruct((N, D), jnp.float32),
        scratch_types=[pltpu.VMEM((bpw,), jnp.int32),
                       pltpu.VMEM((bpw,), jnp.int32),
                       pltpu.VMEM((bpw, D), jnp.float32),
                       pltpu.VMEM((bpw, D), jnp.float32),
                       pltpu.SemaphoreType.DMA,
                       pltpu.SemaphoreType.DMA],
    )
    def k(z_hbm, p0_hbm, p1_hbm, out_hbm, i0_v, i1_v, a_v, b_v, s0, s1):
        wid = lax.axis_index("s") * SC_CORES + lax.axis_index("c")
        base = wid * bpw
        pltpu.sync_copy(p0_hbm.at[pl.ds(base, bpw)], i0_v)
        pltpu.sync_copy(p1_hbm.at[pl.ds(base, bpw)], i1_v)
        c0 = pltpu.async_copy(z_hbm.at[i0_v], a_v, s0)
        c1 = pltpu.async_copy(z_hbm.at[i1_v], b_v, s1)
        c0.wait()
        c1.wait()

        @pl.loop(0, bpw)
        def _(r):
            @pl.loop(0, D, step=16)
            def _(c):
                a_v[r, pl.ds(c, 16)] = a_v[r, pl.ds(c, 16)] + b_v[r, pl.ds(c, 16)]

        pltpu.sync_copy(a_v, out_hbm.at[pl.ds(base, bpw)])

    return k(z, p0, p1)


def kernel(x, gate_w, W1, W2):
    e0, e1, w0, w1 = _gating(x, gate_w)
    md = _route_metadata(e0, e1, w0, w1)
    sorted_x = _sc_dispatch(x, md["tok"])
    z = _gmm(sorted_x, W1, W2, md["sorted_w"],
             md["grp"], md["tile"], md["lo"], md["hi"], md["first"])
    return _sc_combine(z, md["p0"], md["p1"])


# speedup vs baseline: 1.0912x; 1.0912x over previous
import functools

import jax
import jax.numpy as jnp
from jax import lax
from jax.experimental import pallas as pl
from jax.experimental.pallas import tpu as pltpu
from jax.experimental.pallas import tpu_sc as plsc

E = 64
D = 768
F = 3072
N = 2048
K = 2
R = N * K

T = 256
T_LOG2 = 8
NT = R // T
NI = NT + E
NF = 1
FC = F // NF

SC_CORES = 2
SC_SUBCORES = 16
SC_WORKERS = SC_CORES * SC_SUBCORES


def _gating_body(x_ref, gw_ref, e0_ref, e1_ref, w0_ref, w1_ref):
    x = x_ref[...]
    gw = gw_ref[...]
    logits = lax.dot_general(x, gw, (((1,), (1,)), ((), ())),
                             preferred_element_type=jnp.float32)
    col = lax.broadcasted_iota(jnp.int32, logits.shape, 1)
    m1 = jnp.max(logits, axis=1)
    a1 = jnp.min(jnp.where(logits == m1[:, None], col, E), axis=1)
    masked = jnp.where(col == a1[:, None], -1e30, logits)
    m2 = jnp.max(masked, axis=1)
    a2 = jnp.min(jnp.where(masked == m2[:, None], col, E), axis=1)
    e2 = jnp.exp(m2 - m1)
    denom = 1.0 + e2
    e0_ref[...] = a1[:, None]
    e1_ref[...] = a2[:, None]
    w0_ref[...] = (1.0 / denom)[:, None]
    w1_ref[...] = (e2 / denom)[:, None]


def _gating(x, gate_w, interpret=False):
    return pl.pallas_call(
        _gating_body,
        out_shape=(
            jax.ShapeDtypeStruct((N, 1), jnp.int32),
            jax.ShapeDtypeStruct((N, 1), jnp.int32),
            jax.ShapeDtypeStruct((N, 1), jnp.float32),
            jax.ShapeDtypeStruct((N, 1), jnp.float32),
        ),
        interpret=interpret,
    )(x, gate_w)


def _gating_route_body(x_ref, gw_ref, p0_ref, p1_ref, w0_ref, w1_ref,
                       grp_ref, tile_ref, lo_ref, hi_ref, first_ref):
    x = x_ref[...]
    gw = gw_ref[...]
    logits = lax.dot_general(x, gw, (((1,), (1,)), ((), ())),
                             preferred_element_type=jnp.float32)
    col = lax.broadcasted_iota(jnp.int32, logits.shape, 1)
    m1 = jnp.max(logits, axis=1)
    oh0 = logits == m1[:, None]
    a1 = jnp.min(jnp.where(oh0, col, E), axis=1)
    oh0 = col == a1[:, None]
    masked = jnp.where(oh0, -1e30, logits)
    m2 = jnp.max(masked, axis=1)
    a2 = jnp.min(jnp.where(masked == m2[:, None], col, E), axis=1)
    oh1 = col == a2[:, None]
    e2 = jnp.exp(m2 - m1)
    denom = 1.0 + e2
    w0_ref[...] = (1.0 / denom)[:, None]
    w1_ref[...] = (e2 / denom)[:, None]

    f0 = oh0.astype(jnp.float32)
    f1 = oh1.astype(jnp.float32)
    c0, c1 = f0, f1
    k = 1
    while k < N:
        z = jnp.zeros((k, E), jnp.float32)
        c0 = c0 + jnp.concatenate([z, c0[:-k]], axis=0)
        c1 = c1 + jnp.concatenate([z, c1[:-k]], axis=0)
        k *= 2
    count0 = c0[N - 1:N, :]
    counts = count0 + c1[N - 1:N, :]
    er = lax.broadcasted_iota(jnp.int32, (E, E), 0)
    ec = lax.broadcasted_iota(jnp.int32, (E, E), 1)
    tri = (er <= ec).astype(jnp.float32)
    ends = lax.dot_general(counts, tri, (((1,), (0,)), ((), ())),
                           preferred_element_type=jnp.float32)
    starts = ends - counts

    rank0 = jnp.sum(c0 * f0, axis=1) - 1.0
    rank1 = jnp.sum(c1 * f1, axis=1) - 1.0
    pos0 = jnp.sum(starts * f0, axis=1) + rank0
    pos1 = jnp.sum((starts + count0) * f1, axis=1) + rank1
    p0_ref[...] = pos0.astype(jnp.int32)[:, None]
    p1_ref[...] = pos1.astype(jnp.int32)[:, None]

    starts_i = starts.astype(jnp.int32)
    ends_i = ends.astype(jnp.int32)
    counts_i = counts.astype(jnp.int32)
    t0 = lax.shift_right_logical(starts_i, T_LOG2)
    t1 = lax.shift_right_logical(ends_i + (T - 1), T_LOG2)
    nitems = jnp.where(counts_i > 0, t1 - t0, 0).astype(jnp.float32)
    item_cum = lax.dot_general(nitems, tri, (((1,), (0,)), ((), ())),
                               preferred_element_type=jnp.float32)
    item_cum_i = item_cum.astype(jnp.int32)
    ii = lax.broadcasted_iota(jnp.int32, (NI, E), 0)
    e_i = jnp.sum((item_cum_i[0, :][None, :] <= ii).astype(jnp.float32),
                  axis=1).astype(jnp.int32)
    valid = e_i < E
    e_c = jnp.minimum(e_i, E - 1)
    ohe = (lax.broadcasted_iota(jnp.int32, (NI, E), 1) == e_c[:, None]
           ).astype(jnp.float32)
    cum_prev = jnp.concatenate(
        [jnp.zeros((1, 1), jnp.float32), item_cum[:, :-1]], axis=1)
    prev = jnp.sum(ohe * cum_prev, axis=1).astype(jnp.int32)
    t0g = jnp.sum(ohe * t0.astype(jnp.float32), axis=1).astype(jnp.int32)
    sg = jnp.sum(ohe * starts, axis=1).astype(jnp.int32)
    eg = jnp.sum(ohe * ends, axis=1).astype(jnp.int32)
    i_idx = lax.broadcasted_iota(jnp.int32, (NI,), 0)
    tile = t0g + (i_idx - prev)
    tile = jnp.where(valid, tile, NT - 1)
    grp_pad = jnp.max(jnp.where(counts_i[0, :] > 0,
                                lax.broadcasted_iota(jnp.int32, (E,), 0), -1))
    grp = jnp.where(valid, e_c, grp_pad)
    lo = jnp.clip(sg - tile * T, 0, T)
    hi = jnp.clip(eg - tile * T, 0, T)
    lo = jnp.where(valid, lo, 0)
    hi = jnp.where(valid, hi, 0)
    tile2 = tile[:, None]
    prev_tile = jnp.concatenate(
        [jnp.full((1, 1), -1, jnp.int32), tile2[:-1]], axis=0)
    first = (valid[:, None] & (tile2 != prev_tile)).astype(jnp.int32)
    grp_ref[...] = grp[:, None]
    tile_ref[...] = tile2
    lo_ref[...] = lo[:, None]
    hi_ref[...] = hi[:, None]
    first_ref[...] = first


def _gating_route(x, gate_w, interpret=False):
    i32 = jnp.int32
    return pl.pallas_call(
        _gating_route_body,
        out_shape=(
            jax.ShapeDtypeStruct((N, 1), i32),
            jax.ShapeDtypeStruct((N, 1), i32),
            jax.ShapeDtypeStruct((N, 1), jnp.float32),
            jax.ShapeDtypeStruct((N, 1), jnp.float32),
            jax.ShapeDtypeStruct((NI, 1), i32),
            jax.ShapeDtypeStruct((NI, 1), i32),
            jax.ShapeDtypeStruct((NI, 1), i32),
            jax.ShapeDtypeStruct((NI, 1), i32),
            jax.ShapeDtypeStruct((NI, 1), i32),
        ),
        interpret=interpret,
    )(x, gate_w)


def _route_metadata(e0, e1, w0, w1):
    flat_e = jnp.concatenate([e0[:, 0], e1[:, 0]])
    flat_w = jnp.concatenate([w0[:, 0], w1[:, 0]])
    perm = jnp.argsort(flat_e, stable=True).astype(jnp.int32)
    tok = (perm % N).astype(jnp.int32)
    sorted_w = flat_w[perm][:, None]
    inv = jnp.zeros((R,), jnp.int32).at[perm].set(
        jnp.arange(R, dtype=jnp.int32))
    p0, p1 = inv[:N], inv[N:]

    counts = jnp.bincount(flat_e, length=E).astype(jnp.int32)
    ends = jnp.cumsum(counts).astype(jnp.int32)
    starts = ends - counts
    t0 = starts // T
    t1 = (ends + T - 1) // T
    nitems = jnp.where(counts > 0, t1 - t0, 0)
    item_cum = jnp.cumsum(nitems).astype(jnp.int32)
    total = item_cum[E - 1]

    i_idx = jnp.arange(NI, dtype=jnp.int32)
    e_i = jnp.searchsorted(item_cum, i_idx, side="right").astype(jnp.int32)
    valid = i_idx < total
    e_c = jnp.minimum(e_i, E - 1)
    prev = jnp.where(e_c > 0, item_cum[e_c - 1], 0)
    tile = (t0[e_c] + (i_idx - prev)).astype(jnp.int32)
    tile = jnp.where(valid, tile, NT - 1)
    grp = jnp.where(valid, e_c, flat_e[perm[R - 1]]).astype(jnp.int32)
    lo = jnp.clip(starts[e_c] - tile * T, 0, T)
    hi = jnp.clip(ends[e_c] - tile * T, 0, T)
    lo = jnp.where(valid, lo, 0).astype(jnp.int32)
    hi = jnp.where(valid, hi, 0).astype(jnp.int32)
    prev_tile = jnp.concatenate([jnp.full((1,), -1, jnp.int32), tile[:-1]])
    first = (valid & (tile != prev_tile)).astype(jnp.int32)
    return dict(tok=tok, sorted_w=sorted_w, p0=p0, p1=p1,
                grp=grp, tile=tile, lo=lo, hi=hi, first=first)


def _sc_dispatch(x, tok):
    bpw = R // SC_WORKERS
    mesh = plsc.VectorSubcoreMesh(core_axis_name="c", subcore_axis_name="s")

    @functools.partial(
        pl.kernel, mesh=mesh,
        out_type=jax.ShapeDtypeStruct((R, D), jnp.float32),
        scratch_types=[pltpu.VMEM((bpw,), jnp.int32),
                       pltpu.VMEM((bpw, D), jnp.float32),
                       pltpu.SemaphoreType.DMA],
    )
    def k(x_hbm, tok_hbm, out_hbm, idx_v, rows_v, sem):
        wid = lax.axis_index("s") * SC_CORES + lax.axis_index("c")
        base = wid * bpw
        pltpu.sync_copy(tok_hbm.at[pl.ds(base, bpw)], idx_v)
        pltpu.async_copy(x_hbm.at[idx_v], rows_v, sem).wait()
        pltpu.sync_copy(rows_v, out_hbm.at[pl.ds(base, bpw)])

    return k(x, tok)


def _gmm_body(grp_ref, tile_ref, lo_ref, hi_ref, first_ref,
              x_ref, w1_ref, w2_ref, sw_ref, out_ref):
    i = pl.program_id(0)
    j = pl.program_id(1)
    x = x_ref[...]
    h = lax.dot_general(x, w1_ref[0], (((1,), (1,)), ((), ())),
                        preferred_element_type=jnp.float32)
    h = 0.5 * h * (1.0 + lax.erf(h * 0.7071067811865476))
    z = lax.dot_general(h, w2_ref[0], (((1,), (1,)), ((), ())),
                        preferred_element_type=jnp.float32)
    z = z * sw_ref[...]
    rows = lax.broadcasted_iota(jnp.int32, (T, 1), 0)
    mask = (rows >= lo_ref[i]) & (rows < hi_ref[i])
    z = jnp.where(mask, z, 0.0)
    is_first = (first_ref[i] == 1) & (j == 0)

    @pl.when(is_first)
    def _():
        out_ref[...] = z

    @pl.when(jnp.logical_not(is_first))
    def _():
        out_ref[...] += z


def _gmm(sorted_x, W1, W2, sorted_w, grp, tile, lo, hi, first, interpret=False):
    grid_spec = pltpu.PrefetchScalarGridSpec(
        num_scalar_prefetch=5,
        grid=(NI, NF),
        in_specs=[
            pl.BlockSpec((T, D), lambda i, j, g, t, *_: (t[i], 0)),
            pl.BlockSpec((1, FC, D), lambda i, j, g, t, *_: (g[i], j, 0)),
            pl.BlockSpec((1, D, FC), lambda i, j, g, t, *_: (g[i], 0, j)),
            pl.BlockSpec((T, 1), lambda i, j, g, t, *_: (t[i], 0)),
        ],
        out_specs=pl.BlockSpec((T, D), lambda i, j, g, t, *_: (t[i], 0)),
    )
    return pl.pallas_call(
        _gmm_body,
        grid_spec=grid_spec,
        out_shape=jax.ShapeDtypeStruct((R, D), jnp.float32),
        compiler_params=pltpu.CompilerParams(
            dimension_semantics=("arbitrary", "arbitrary")),
        interpret=interpret,
    )(grp, tile, lo, hi, first, sorted_x, W1, W2, sorted_w)


def _sc_combine(z, p0, p1):
    bpw = N // SC_WORKERS
    mesh = plsc.VectorSubcoreMesh(core_axis_name="c", subcore_axis_name="s")

    @functools.partial(
        pl.kernel, mesh=mesh,
        out_type=jax.ShapeDtypeStruct((N, D), jnp.float32),
        scratch_types=[pltpu.VMEM((bpw,), jnp.int32),
                       pltpu.VMEM((bpw,), jnp.int32),
                       pltpu.VMEM((bpw, D), jnp.float32),
                       pltpu.VMEM((bpw, D), jnp.float32),
                       pltpu.SemaphoreType.DMA,
                       pltpu.SemaphoreType.DMA],
    )
    def k(z_hbm, p0_hbm, p1_hbm, out_hbm, i0_v, i1_v, a_v, b_v, s0, s1):
        wid = lax.axis_index("s") * SC_CORES + lax.axis_index("c")
        base = wid * bpw
        pltpu.sync_copy(p0_hbm.at[pl.ds(base, bpw)], i0_v)
        pltpu.sync_copy(p1_hbm.at[pl.ds(base, bpw)], i1_v)
        c0 = pltpu.async_copy(z_hbm.at[i0_v], a_v, s0)
        c1 = pltpu.async_copy(z_hbm.at[i1_v], b_v, s1)
        c0.wait()
        c1.wait()

        @pl.loop(0, bpw)
        def _(r):
            @pl.loop(0, D, step=16)
            def _(c):
                a_v[r, pl.ds(c, 16)] = a_v[r, pl.ds(c, 16)] + b_v[r, pl.ds(c, 16)]

        pltpu.sync_copy(a_v, out_hbm.at[pl.ds(base, bpw)])

    return k(z, p0, p1)


def _glue(p0, p1, w0, w1):
    p0f, p1f = p0[:, 0], p1[:, 0]
    pr = jnp.concatenate([p0f, p1f])
    ar = jnp.arange(N, dtype=jnp.int32)
    tok = jnp.zeros((R,), jnp.int32).at[pr].set(jnp.concatenate([ar, ar]))
    sorted_w = jnp.zeros((R,), jnp.float32).at[pr].set(
        jnp.concatenate([w0[:, 0], w1[:, 0]]))[:, None]
    return p0f, p1f, tok, sorted_w


def kernel(x, gate_w, W1, W2):
    p0, p1, w0, w1, grp, tile, lo, hi, first = _gating_route(x, gate_w)
    p0f, p1f, tok, sorted_w = _glue(p0, p1, w0, w1)
    sorted_x = _sc_dispatch(x, tok)
    z = _gmm(sorted_x, W1, W2, sorted_w,
             grp[:, 0], tile[:, 0], lo[:, 0], hi[:, 0], first[:, 0])
    return _sc_combine(z, p0f, p1f)
